# baseline (speedup 1.0000x reference)
import functools

import jax
import jax.numpy as jnp
from jax import lax
from jax.experimental import pallas as pl
from jax.experimental.pallas import tpu as pltpu
from jax.experimental.pallas import tpu_sc as plsc

N_NODES = 10000
N_EDGES = 320000
N_GRAPHS = 64
D = 128

NC = 2
NS = 16
NW = NC * NS
EB = 128
NIT = 80
EPAD = NW * NIT * EB
NPAD = 10240
RPT = NPAD // NS

_mesh = plsc.VectorSubcoreMesh(core_axis_name="c", subcore_axis_name="s")
_sc_params = pltpu.CompilerParams(needs_layout_passes=False)


@functools.partial(
    pl.kernel,
    out_type=jax.ShapeDtypeStruct((NW, NPAD), jnp.float32),
    mesh=_mesh,
    scratch_types=[
        pltpu.VMEM((NIT * EB,), jnp.int32),
        pltpu.VMEM((NPAD,), jnp.float32),
    ],
    compiler_params=_sc_params,
)
def _sc_counts(dst_hbm, out_hbm, dst_v, cnt_v):
    c = lax.axis_index("c")
    s = lax.axis_index("s")
    wid = s * NC + c
    pltpu.sync_copy(dst_hbm.at[wid], dst_v)

    def zero(i, carry):
        cnt_v[pl.ds(i * 16, 16)] = jnp.zeros((16,), jnp.float32)
        return carry

    lax.fori_loop(0, NPAD // 16, zero, 0)

    ones = jnp.ones((16,), jnp.float32)

    def body(k, carry):
        idx = dst_v[pl.ds(k * 16, 16)]
        plsc.addupdate_scatter(cnt_v, [idx], ones)
        return carry

    lax.fori_loop(0, (NIT * EB) // 16, body, 0)
    pltpu.sync_copy(cnt_v, out_hbm.at[wid])


NBUF = 2
WIN = 16
NIT_A = 160
NWIN = NIT_A // WIN
TARGET_CORE = 0


@functools.partial(
    pl.kernel,
    out_type=jax.ShapeDtypeStruct((NC, NPAD, D), jnp.float32),
    mesh=_mesh,
    scratch_types=[
        pltpu.VMEM((NBUF, WIN, EB), jnp.int32),
        pltpu.VMEM((NBUF, WIN, EB), jnp.int32),
        pltpu.VMEM((EB, D), jnp.float32),
        pltpu.VMEM((EB, D), jnp.float32),
        pltpu.SemaphoreType.DMA,
        pltpu.SemaphoreType.DMA,
        pltpu.SemaphoreType.DMA,
        pltpu.SemaphoreType.DMA,
        pltpu.VMEM_SHARED((NPAD, D), jnp.float32),
    ],
    compiler_params=_sc_params,
)
def _sc_agg(src_hbm, dst_hbm, table_hbm, out_hbm, src_w, dst_w,
            buf0, buf1, sem0, sem1, isem_s, isem_d, acc):
    bufs = (buf0, buf1)
    sems = (sem0, sem1)
    c = lax.axis_index("c")
    s = lax.axis_index("s")
    wid = s

    def zero(i, carry):
        for cc in range(D // 16):
            buf0[i, pl.ds(cc * 16, 16)] = jnp.zeros((16,), jnp.float32)
        return carry

    lax.fori_loop(0, EB, zero, 0)
    for t in range(RPT // EB):
        pltpu.sync_copy(buf0, acc.at[pl.ds(s * RPT + t * EB, EB)])
    plsc.subcore_barrier()

    @pl.when(c == TARGET_CORE)
    def _edge_phase():
        pltpu.sync_copy(src_hbm.at[wid, pl.ds(0, WIN)], src_w.at[0])
        pltpu.sync_copy(dst_hbm.at[wid, pl.ds(0, WIN)], dst_w.at[0])
        for w in range(NWIN):
            p = w % 2
            if w + 1 < NWIN:
                np_ = (w + 1) % 2
                pltpu.async_copy(src_hbm.at[wid, pl.ds((w + 1) * WIN, WIN)],
                                 src_w.at[np_], isem_s)
                pltpu.async_copy(dst_hbm.at[wid, pl.ds((w + 1) * WIN, WIN)],
                                 dst_w.at[np_], isem_d)
            for b in range(NBUF):
                pltpu.async_copy(table_hbm.at[src_w.at[p, b]], bufs[b], sems[b])

            def body(g, carry, p=p):
                for b in range(NBUF):
                    j = g * NBUF + b
                    pltpu.make_async_copy(table_hbm.at[src_w.at[p, j]],
                                          bufs[b], sems[b]).wait()
                    pltpu.sync_copy(bufs[b], acc.at[dst_w.at[p, j]], add=True)
                    nj = j + NBUF

                    @pl.when(nj < WIN)
                    def _():
                        pltpu.async_copy(table_hbm.at[src_w.at[p, nj]],
                                         bufs[b], sems[b])
                return carry

            lax.fori_loop(0, WIN // NBUF, body, 0)
            if w + 1 < NWIN:
                np_ = (w + 1) % 2
                pltpu.make_async_copy(src_hbm.at[wid, pl.ds((w + 1) * WIN, WIN)],
                                      src_w.at[np_], isem_s).wait()
                pltpu.make_async_copy(dst_hbm.at[wid, pl.ds((w + 1) * WIN, WIN)],
                                      dst_w.at[np_], isem_d).wait()

    plsc.subcore_barrier()
    pltpu.sync_copy(acc.at[pl.ds(s * RPT, RPT)], out_hbm.at[c, pl.ds(s * RPT, RPT)])


RB = 1000
NG = N_NODES // RB


def _mm(a, b):
    return jnp.dot(a, b, preferred_element_type=jnp.float32)


def _row_spec():
    return pl.BlockSpec((RB, D), lambda i: (i, 0))


def _p_spec():
    return pl.BlockSpec((NC, RB, D), lambda i: (0, i, 0))


def _dis_spec():
    return pl.BlockSpec((RB, 1), lambda i: (i, 0))


def _full_spec(shape):
    nd = len(shape)
    return pl.BlockSpec(shape, lambda i: (0,) * nd)


def _dis_body(cnt_ref, out_ref):
    total = jnp.sum(cnt_ref[...], axis=0)
    out_ref[...] = lax.rsqrt(1.0 + total)


def _dis_call(counts):
    return pl.pallas_call(
        _dis_body, out_shape=jax.ShapeDtypeStruct((NPAD,), jnp.float32))(counts)


def _hs1_body(x_ref, w_ref, dis_ref, out_ref):
    out_ref[...] = _mm(x_ref[...], w_ref[...]) * dis_ref[...]


def _hs1_call(x, W1, dis_col):
    return pl.pallas_call(
        _hs1_body,
        grid=(NG,),
        in_specs=[_row_spec(), _full_spec((D, D)), _dis_spec()],
        out_specs=_row_spec(),
        out_shape=jax.ShapeDtypeStruct((N_NODES, D), jnp.float32),
    )(x, W1, dis_col)


def _layer2_body(p_ref, hs_ref, dis_ref, b_ref, wa_ref, wb_ref, oa_ref, ob_ref):
    dis = dis_ref[...]
    agg = p_ref[0] + p_ref[1] + hs_ref[...]
    h = jnp.maximum(agg * dis + b_ref[...], 0.0)
    oa_ref[...] = _mm(h, wa_ref[...]) * dis
    ob_ref[...] = _mm(h, wb_ref[...]) * dis


def _layer2_call(p, hs1, dis_col, b1, W2a, W2b):
    return pl.pallas_call(
        _layer2_body,
        grid=(NG,),
        in_specs=[_p_spec(), _row_spec(), _dis_spec(), _full_spec((1, D)),
                  _full_spec((D, D)), _full_spec((D, D))],
        out_specs=[_row_spec(), _row_spec()],
        out_shape=[jax.ShapeDtypeStruct((N_NODES, D), jnp.float32)] * 2,
    )(p, hs1, dis_col, b1, W2a, W2b)


def _layer3_body(pa_ref, pb_ref, hsa_ref, hsb_ref, dis_ref, ba_ref, bb_ref,
                 wa_ref, wb_ref, out_ref):
    dis = dis_ref[...]
    ha = jnp.maximum((pa_ref[0] + pa_ref[1] + hsa_ref[...]) * dis
                     + ba_ref[...], 0.0)
    hb = jnp.maximum((pb_ref[0] + pb_ref[1] + hsb_ref[...]) * dis
                     + bb_ref[...], 0.0)
    out_ref[...] = (_mm(ha, wa_ref[...]) + _mm(hb, wb_ref[...])) * dis


def _layer3_call(pa, pb, hs2a, hs2b, dis_col, b2a, b2b, W3a, W3b):
    return pl.pallas_call(
        _layer3_body,
        grid=(NG,),
        in_specs=[_p_spec(), _p_spec(), _row_spec(), _row_spec(), _dis_spec(),
                  _full_spec((1, D)), _full_spec((1, D)),
                  _full_spec((D, D)), _full_spec((D, D))],
        out_specs=_row_spec(),
        out_shape=jax.ShapeDtypeStruct((N_NODES, D), jnp.float32),
    )(pa, pb, hs2a, hs2b, dis_col, b2a, b2b, W3a, W3b)


def _head_body(p_ref, hs_ref, dis_ref, b_ref, batch_ref, wf1_ref, bf1_ref,
               wf2_ref, bf2_ref, out_ref, seg_acc, cnt_acc):
    i = pl.program_id(0)

    @pl.when(i == 0)
    def _():
        seg_acc[...] = jnp.zeros_like(seg_acc)
        cnt_acc[...] = jnp.zeros_like(cnt_acc)

    dis = dis_ref[...]
    h3 = jnp.maximum((p_ref[0] + p_ref[1] + hs_ref[...]) * dis
                     + b_ref[...], 0.0)
    gids = lax.broadcasted_iota(jnp.int32, (RB, N_GRAPHS), 1)
    onehot_t = (gids == batch_ref[...]).astype(jnp.float32)
    seg_acc[...] += lax.dot_general(
        onehot_t, h3, (((0,), (0,)), ((), ())),
        precision=lax.Precision.HIGHEST,
        preferred_element_type=jnp.float32)
    cnt_acc[...] += jnp.sum(onehot_t, axis=0)[:, None]

    @pl.when(i == NG - 1)
    def _():
        pooled = seg_acc[...] / jnp.maximum(cnt_acc[...], 1.0)
        o1 = jnp.maximum(_mm(pooled, wf1_ref[...]) + bf1_ref[...], 0.0)
        out_ref[...] = _mm(o1, wf2_ref[...]) + bf2_ref[...]


def _head_call(p3, hs3, dis_col, b3, batch2d, Wf1, bf1, Wf2, bf2):
    return pl.pallas_call(
        _head_body,
        grid=(NG,),
        in_specs=[_p_spec(), _row_spec(), _dis_spec(), _full_spec((1, D)),
                  pl.BlockSpec((RB, 1), lambda i: (i, 0)),
                  _full_spec((D, 64)), _full_spec((1, 64)),
                  _full_spec((64, 1)), _full_spec((1, 1))],
        out_specs=pl.BlockSpec((N_GRAPHS, 1), lambda i: (0, 0)),
        out_shape=jax.ShapeDtypeStruct((N_GRAPHS, 1), jnp.float32),
        scratch_shapes=[pltpu.VMEM((N_GRAPHS, D), jnp.float32),
                        pltpu.VMEM((N_GRAPHS, 1), jnp.float32)],
    )(p3, hs3, dis_col, b3, batch2d, Wf1, bf1, Wf2, bf2)


def kernel(x, edge_index, batch, W1, b1, W2, b2, W3, b3, Wf1, bf1, Wf2, bf2):
    f32 = jnp.float32
    src = edge_index[0].astype(jnp.int32)
    dst = edge_index[1].astype(jnp.int32)
    pad = EPAD - N_EDGES
    srcp = jnp.concatenate([src, jnp.zeros((pad,), jnp.int32)])
    pad_dst = N_NODES + jnp.arange(pad, dtype=jnp.int32) % (NPAD - N_NODES)
    dstp = jnp.concatenate([dst, pad_dst])
    src3 = srcp.reshape(NS, NIT_A, EB)
    dst3 = dstp.reshape(NS, NIT_A, EB)
    dst2 = dstp.reshape(NW, EPAD // NW)

    counts = _sc_counts(dst2)
    dis_flat = _dis_call(counts)
    dis_col = dis_flat.reshape(NPAD, 1)[:N_NODES]

    hs1 = _hs1_call(x, W1, dis_col)
    p1 = _sc_agg(src3, dst3, hs1)
    hs2a, hs2b = _layer2_call(p1, hs1, dis_col, b1.reshape(1, D),
                              W2[:, :D], W2[:, D:])

    p2a = _sc_agg(src3, dst3, hs2a)
    p2b = _sc_agg(src3, dst3, hs2b)
    hs3 = _layer3_call(p2a, p2b, hs2a, hs2b, dis_col,
                       b2[:D].reshape(1, D), b2[D:].reshape(1, D),
                       W3[:D, :], W3[D:, :])

    p3 = _sc_agg(src3, dst3, hs3)
    out = _head_call(p3, hs3, dis_col, b3.reshape(1, D),
                     batch.astype(jnp.int32).reshape(N_NODES, 1),
                     Wf1, bf1.reshape(1, 64), Wf2, bf2.reshape(1, 1))
    return out


# speedup vs baseline: 1.0013x; 1.0013x over previous
import functools

import jax
import jax.numpy as jnp
from jax import lax
from jax.experimental import pallas as pl
from jax.experimental.pallas import tpu as pltpu
from jax.experimental.pallas import tpu_sc as plsc

N_NODES = 10000
N_EDGES = 320000
N_GRAPHS = 64
D = 128

NC = 2
NS = 16
NW = NC * NS
EB = 128
NIT = 80
EPAD = NW * NIT * EB
NPAD = 10240
RPT = NPAD // NS

_mesh = plsc.VectorSubcoreMesh(core_axis_name="c", subcore_axis_name="s")
_sc_params = pltpu.CompilerParams(needs_layout_passes=False)


@functools.partial(
    pl.kernel,
    out_type=jax.ShapeDtypeStruct((NW, NPAD), jnp.float32),
    mesh=_mesh,
    scratch_types=[
        pltpu.VMEM((NIT * EB,), jnp.int32),
        pltpu.VMEM((NPAD,), jnp.float32),
    ],
    compiler_params=_sc_params,
)
def _sc_counts(dst_hbm, out_hbm, dst_v, cnt_v):
    c = lax.axis_index("c")
    s = lax.axis_index("s")
    wid = s * NC + c
    pltpu.sync_copy(dst_hbm.at[wid], dst_v)

    def zero(i, carry):
        cnt_v[pl.ds(i * 16, 16)] = jnp.zeros((16,), jnp.float32)
        return carry

    lax.fori_loop(0, NPAD // 16, zero, 0)

    ones = jnp.ones((16,), jnp.float32)

    def body(k, carry):
        idx = dst_v[pl.ds(k * 16, 16)]
        plsc.addupdate_scatter(cnt_v, [idx], ones)
        return carry

    lax.fori_loop(0, (NIT * EB) // 16, body, 0)
    pltpu.sync_copy(cnt_v, out_hbm.at[wid])


NBUF = 2
WIN = 16
NIT_A = 160
NWIN = NIT_A // WIN
TARGET_CORE = 1


@functools.partial(
    pl.kernel,
    out_type=jax.ShapeDtypeStruct((NC, NPAD, D), jnp.float32),
    mesh=_mesh,
    scratch_types=[
        pltpu.VMEM((NBUF, WIN, EB), jnp.int32),
        pltpu.VMEM((NBUF, WIN, EB), jnp.int32),
        pltpu.VMEM((EB, D), jnp.float32),
        pltpu.VMEM((EB, D), jnp.float32),
        pltpu.SemaphoreType.DMA,
        pltpu.SemaphoreType.DMA,
        pltpu.SemaphoreType.DMA,
        pltpu.SemaphoreType.DMA,
        pltpu.VMEM_SHARED((NPAD, D), jnp.float32),
    ],
    compiler_params=_sc_params,
)
def _sc_agg(src_hbm, dst_hbm, table_hbm, out_hbm, src_w, dst_w,
            buf0, buf1, sem0, sem1, isem_s, isem_d, acc):
    bufs = (buf0, buf1)
    sems = (sem0, sem1)
    c = lax.axis_index("c")
    s = lax.axis_index("s")
    wid = s

    def zero(i, carry):
        for cc in range(D // 16):
            buf0[i, pl.ds(cc * 16, 16)] = jnp.zeros((16,), jnp.float32)
        return carry

    lax.fori_loop(0, EB, zero, 0)
    for t in range(RPT // EB):
        pltpu.sync_copy(buf0, acc.at[pl.ds(s * RPT + t * EB, EB)])
    plsc.subcore_barrier()

    @pl.when(c == TARGET_CORE)
    def _edge_phase():
        pltpu.sync_copy(src_hbm.at[wid, pl.ds(0, WIN)], src_w.at[0])
        pltpu.sync_copy(dst_hbm.at[wid, pl.ds(0, WIN)], dst_w.at[0])
        for w in range(NWIN):
            p = w % 2
            if w + 1 < NWIN:
                np_ = (w + 1) % 2
                pltpu.async_copy(src_hbm.at[wid, pl.ds((w + 1) * WIN, WIN)],
                                 src_w.at[np_], isem_s)
                pltpu.async_copy(dst_hbm.at[wid, pl.ds((w + 1) * WIN, WIN)],
                                 dst_w.at[np_], isem_d)
            for b in range(NBUF):
                pltpu.async_copy(table_hbm.at[src_w.at[p, b]], bufs[b], sems[b])

            def body(g, carry, p=p):
                for b in range(NBUF):
                    j = g * NBUF + b
                    pltpu.make_async_copy(table_hbm.at[src_w.at[p, j]],
                                          bufs[b], sems[b]).wait()
                    pltpu.sync_copy(bufs[b], acc.at[dst_w.at[p, j]], add=True)
                    nj = j + NBUF

                    @pl.when(nj < WIN)
                    def _():
                        pltpu.async_copy(table_hbm.at[src_w.at[p, nj]],
                                         bufs[b], sems[b])
                return carry

            lax.fori_loop(0, WIN // NBUF, body, 0)
            if w + 1 < NWIN:
                np_ = (w + 1) % 2
                pltpu.make_async_copy(src_hbm.at[wid, pl.ds((w + 1) * WIN, WIN)],
                                      src_w.at[np_], isem_s).wait()
                pltpu.make_async_copy(dst_hbm.at[wid, pl.ds((w + 1) * WIN, WIN)],
                                      dst_w.at[np_], isem_d).wait()

    plsc.subcore_barrier()
    pltpu.sync_copy(acc.at[pl.ds(s * RPT, RPT)], out_hbm.at[c, pl.ds(s * RPT, RPT)])


RB = 1000
NG = N_NODES // RB


def _mm(a, b):
    return jnp.dot(a, b, preferred_element_type=jnp.float32)


def _row_spec():
    return pl.BlockSpec((RB, D), lambda i: (i, 0))


def _p_spec():
    return pl.BlockSpec((NC, RB, D), lambda i: (0, i, 0))


def _dis_spec():
    return pl.BlockSpec((RB, 1), lambda i: (i, 0))


def _full_spec(shape):
    nd = len(shape)
    return pl.BlockSpec(shape, lambda i: (0,) * nd)


def _dis_body(cnt_ref, out_ref):
    total = jnp.sum(cnt_ref[...], axis=0)
    out_ref[...] = lax.rsqrt(1.0 + total)


def _dis_call(counts):
    return pl.pallas_call(
        _dis_body, out_shape=jax.ShapeDtypeStruct((NPAD,), jnp.float32))(counts)


def _hs1_body(x_ref, w_ref, dis_ref, out_ref):
    out_ref[...] = _mm(x_ref[...], w_ref[...]) * dis_ref[...]


def _hs1_call(x, W1, dis_col):
    return pl.pallas_call(
        _hs1_body,
        grid=(NG,),
        in_specs=[_row_spec(), _full_spec((D, D)), _dis_spec()],
        out_specs=_row_spec(),
        out_shape=jax.ShapeDtypeStruct((N_NODES, D), jnp.float32),
    )(x, W1, dis_col)


def _layer2_body(p_ref, hs_ref, dis_ref, b_ref, wa_ref, wb_ref, oa_ref, ob_ref):
    dis = dis_ref[...]
    agg = p_ref[0] + p_ref[1] + hs_ref[...]
    h = jnp.maximum(agg * dis + b_ref[...], 0.0)
    oa_ref[...] = _mm(h, wa_ref[...]) * dis
    ob_ref[...] = _mm(h, wb_ref[...]) * dis


def _layer2_call(p, hs1, dis_col, b1, W2a, W2b):
    return pl.pallas_call(
        _layer2_body,
        grid=(NG,),
        in_specs=[_p_spec(), _row_spec(), _dis_spec(), _full_spec((1, D)),
                  _full_spec((D, D)), _full_spec((D, D))],
        out_specs=[_row_spec(), _row_spec()],
        out_shape=[jax.ShapeDtypeStruct((N_NODES, D), jnp.float32)] * 2,
    )(p, hs1, dis_col, b1, W2a, W2b)


def _layer3_body(pa_ref, pb_ref, hsa_ref, hsb_ref, dis_ref, ba_ref, bb_ref,
                 wa_ref, wb_ref, out_ref):
    dis = dis_ref[...]
    ha = jnp.maximum((pa_ref[0] + pa_ref[1] + hsa_ref[...]) * dis
                     + ba_ref[...], 0.0)
    hb = jnp.maximum((pb_ref[0] + pb_ref[1] + hsb_ref[...]) * dis
                     + bb_ref[...], 0.0)
    out_ref[...] = (_mm(ha, wa_ref[...]) + _mm(hb, wb_ref[...])) * dis


def _layer3_call(pa, pb, hs2a, hs2b, dis_col, b2a, b2b, W3a, W3b):
    return pl.pallas_call(
        _layer3_body,
        grid=(NG,),
        in_specs=[_p_spec(), _p_spec(), _row_spec(), _row_spec(), _dis_spec(),
                  _full_spec((1, D)), _full_spec((1, D)),
                  _full_spec((D, D)), _full_spec((D, D))],
        out_specs=_row_spec(),
        out_shape=jax.ShapeDtypeStruct((N_NODES, D), jnp.float32),
    )(pa, pb, hs2a, hs2b, dis_col, b2a, b2b, W3a, W3b)


def _head_body(p_ref, hs_ref, dis_ref, b_ref, batch_ref, wf1_ref, bf1_ref,
               wf2_ref, bf2_ref, out_ref, seg_acc, cnt_acc):
    i = pl.program_id(0)

    @pl.when(i == 0)
    def _():
        seg_acc[...] = jnp.zeros_like(seg_acc)
        cnt_acc[...] = jnp.zeros_like(cnt_acc)

    dis = dis_ref[...]
    h3 = jnp.maximum((p_ref[0] + p_ref[1] + hs_ref[...]) * dis
                     + b_ref[...], 0.0)
    gids = lax.broadcasted_iota(jnp.int32, (RB, N_GRAPHS), 1)
    onehot_t = (gids == batch_ref[...]).astype(jnp.float32)
    seg_acc[...] += lax.dot_general(
        onehot_t, h3, (((0,), (0,)), ((), ())),
        precision=lax.Precision.HIGHEST,
        preferred_element_type=jnp.float32)
    cnt_acc[...] += jnp.sum(onehot_t, axis=0)[:, None]

    @pl.when(i == NG - 1)
    def _():
        pooled = seg_acc[...] / jnp.maximum(cnt_acc[...], 1.0)
        o1 = jnp.maximum(_mm(pooled, wf1_ref[...]) + bf1_ref[...], 0.0)
        out_ref[...] = _mm(o1, wf2_ref[...]) + bf2_ref[...]


def _head_call(p3, hs3, dis_col, b3, batch2d, Wf1, bf1, Wf2, bf2):
    return pl.pallas_call(
        _head_body,
        grid=(NG,),
        in_specs=[_p_spec(), _row_spec(), _dis_spec(), _full_spec((1, D)),
                  pl.BlockSpec((RB, 1), lambda i: (i, 0)),
                  _full_spec((D, 64)), _full_spec((1, 64)),
                  _full_spec((64, 1)), _full_spec((1, 1))],
        out_specs=pl.BlockSpec((N_GRAPHS, 1), lambda i: (0, 0)),
        out_shape=jax.ShapeDtypeStruct((N_GRAPHS, 1), jnp.float32),
        scratch_shapes=[pltpu.VMEM((N_GRAPHS, D), jnp.float32),
                        pltpu.VMEM((N_GRAPHS, 1), jnp.float32)],
    )(p3, hs3, dis_col, b3, batch2d, Wf1, bf1, Wf2, bf2)


def kernel(x, edge_index, batch, W1, b1, W2, b2, W3, b3, Wf1, bf1, Wf2, bf2):
    f32 = jnp.float32
    src = edge_index[0].astype(jnp.int32)
    dst = edge_index[1].astype(jnp.int32)
    pad = EPAD - N_EDGES
    srcp = jnp.concatenate([src, jnp.zeros((pad,), jnp.int32)])
    pad_dst = N_NODES + jnp.arange(pad, dtype=jnp.int32) % (NPAD - N_NODES)
    dstp = jnp.concatenate([dst, pad_dst])
    src3 = srcp.reshape(NS, NIT_A, EB)
    dst3 = dstp.reshape(NS, NIT_A, EB)
    dst2 = dstp.reshape(NW, EPAD // NW)

    counts = _sc_counts(dst2)
    dis_flat = _dis_call(counts)
    dis_col = dis_flat.reshape(NPAD, 1)[:N_NODES]

    hs1 = _hs1_call(x, W1, dis_col)
    p1 = _sc_agg(src3, dst3, hs1)
    hs2a, hs2b = _layer2_call(p1, hs1, dis_col, b1.reshape(1, D),
                              W2[:, :D], W2[:, D:])

    p2a = _sc_agg(src3, dst3, hs2a)
    p2b = _sc_agg(src3, dst3, hs2b)
    hs3 = _layer3_call(p2a, p2b, hs2a, hs2b, dis_col,
                       b2[:D].reshape(1, D), b2[D:].reshape(1, D),
                       W3[:D, :], W3[D:, :])

    p3 = _sc_agg(src3, dst3, hs3)
    out = _head_call(p3, hs3, dis_col, b3.reshape(1, D),
                     batch.astype(jnp.int32).reshape(N_NODES, 1),
                     Wf1, bf1.reshape(1, 64), Wf2, bf2.reshape(1, 1))
    return out


# speedup vs baseline: 1.1042x; 1.1029x over previous
import functools

import jax
import jax.numpy as jnp
from jax import lax
from jax.experimental import pallas as pl
from jax.experimental.pallas import tpu as pltpu
from jax.experimental.pallas import tpu_sc as plsc

N_NODES = 10000
N_EDGES = 320000
N_GRAPHS = 64
D = 128

NC = 2
NS = 16
NW = NC * NS
EB = 128
NIT = 80
EPAD = NW * NIT * EB
NPAD = 10240
RPT = NPAD // NS

_mesh = plsc.VectorSubcoreMesh(core_axis_name="c", subcore_axis_name="s")
_sc_params = pltpu.CompilerParams(needs_layout_passes=False)


@functools.partial(
    pl.kernel,
    out_type=jax.ShapeDtypeStruct((NW, NPAD), jnp.float32),
    mesh=_mesh,
    scratch_types=[
        pltpu.VMEM((NIT * EB,), jnp.int32),
        pltpu.VMEM((NPAD,), jnp.float32),
    ],
    compiler_params=_sc_params,
)
def _sc_counts(dst_hbm, out_hbm, dst_v, cnt_v):
    c = lax.axis_index("c")
    s = lax.axis_index("s")
    wid = s * NC + c
    pltpu.sync_copy(dst_hbm.at[wid], dst_v)

    def zero(i, carry):
        cnt_v[pl.ds(i * 16, 16)] = jnp.zeros((16,), jnp.float32)
        return carry

    lax.fori_loop(0, NPAD // 16, zero, 0)

    ones = jnp.ones((16,), jnp.float32)

    def body(k, carry):
        idx = dst_v[pl.ds(k * 16, 16)]
        plsc.addupdate_scatter(cnt_v, [idx], ones)
        return carry

    lax.fori_loop(0, (NIT * EB) // 16, body, 0)
    pltpu.sync_copy(cnt_v, out_hbm.at[wid])


NBUF = 2
WIN = 16
NWIN = NIT // WIN


@functools.partial(
    pl.kernel,
    out_type=jax.ShapeDtypeStruct((NC, NPAD, D), jnp.float32),
    mesh=_mesh,
    scratch_types=[
        pltpu.VMEM((2, WIN, EB), jnp.int32),
        pltpu.VMEM((2, WIN, EB), jnp.int32),
        pltpu.VMEM((EB, D), jnp.float32),
        pltpu.VMEM((EB, D), jnp.float32),
        pltpu.SemaphoreType.DMA,
        pltpu.SemaphoreType.DMA,
        pltpu.SemaphoreType.DMA,
        pltpu.SemaphoreType.DMA,
        pltpu.SemaphoreType.DMA,
        pltpu.SemaphoreType.DMA,
        pltpu.VMEM_SHARED((NPAD, D), jnp.float32),
    ],
    compiler_params=_sc_params,
)
def _sc_agg(src_hbm, dst_hbm, table_hbm, out_hbm, src_w, dst_w,
            buf0, buf1, sem0, sem1, isem_s, isem_d, ssem0, ssem1, acc):
    bufs = (buf0, buf1)
    sems = (sem0, sem1)
    c = lax.axis_index("c")
    s = lax.axis_index("s")
    wid = s * NC + c

    pltpu.sync_copy(src_hbm.at[wid, pl.ds(0, WIN)], src_w.at[0])
    pltpu.sync_copy(dst_hbm.at[wid, pl.ds(0, WIN)], dst_w.at[0])

    def zero(i, carry):
        for cc in range(D // 16):
            buf0[i, pl.ds(cc * 16, 16)] = jnp.zeros((16,), jnp.float32)
        return carry

    lax.fori_loop(0, EB, zero, 0)
    for t in range(RPT // EB):
        pltpu.sync_copy(buf0, acc.at[pl.ds(s * RPT + t * EB, EB)])
    plsc.subcore_barrier()

    for w in range(NWIN):
        p = w % 2
        if w + 1 < NWIN:
            np_ = (w + 1) % 2
            pltpu.async_copy(src_hbm.at[wid, pl.ds((w + 1) * WIN, WIN)],
                             src_w.at[np_], isem_s)
            pltpu.async_copy(dst_hbm.at[wid, pl.ds((w + 1) * WIN, WIN)],
                             dst_w.at[np_], isem_d)
        for b in range(NBUF):
            pltpu.async_copy(table_hbm.at[src_w.at[p, b]], bufs[b], sems[b])

        def body(g, carry, p=p):
            ssems = (ssem0, ssem1)
            scs = []
            for b in range(NBUF):
                j = g * NBUF + b
                pltpu.make_async_copy(table_hbm.at[src_w.at[p, j]],
                                      bufs[b], sems[b]).wait()
                scs.append(pltpu.async_copy(bufs[b], acc.at[dst_w.at[p, j]],
                                            ssems[b], add=True))
            for b in range(NBUF):
                j = g * NBUF + b
                scs[b].wait()
                nj = j + NBUF

                @pl.when(nj < WIN)
                def _():
                    pltpu.async_copy(table_hbm.at[src_w.at[p, nj]],
                                     bufs[b], sems[b])
            return carry

        lax.fori_loop(0, WIN // NBUF, body, 0)
        if w + 1 < NWIN:
            np_ = (w + 1) % 2
            pltpu.make_async_copy(src_hbm.at[wid, pl.ds((w + 1) * WIN, WIN)],
                                  src_w.at[np_], isem_s).wait()
            pltpu.make_async_copy(dst_hbm.at[wid, pl.ds((w + 1) * WIN, WIN)],
                                  dst_w.at[np_], isem_d).wait()

    plsc.subcore_barrier()
    pltpu.sync_copy(acc.at[pl.ds(s * RPT, RPT)], out_hbm.at[c, pl.ds(s * RPT, RPT)])


RB = 1000
NG = N_NODES // RB


def _mm(a, b):
    return jnp.dot(a, b, preferred_element_type=jnp.float32)


def _row_spec():
    return pl.BlockSpec((RB, D), lambda i: (i, 0))


def _p_spec():
    return pl.BlockSpec((NC, RB, D), lambda i: (0, i, 0))


def _dis_spec():
    return pl.BlockSpec((RB, 1), lambda i: (i, 0))


def _full_spec(shape):
    nd = len(shape)
    return pl.BlockSpec(shape, lambda i: (0,) * nd)


def _dis_body(cnt_ref, out_ref):
    total = jnp.sum(cnt_ref[...], axis=0)
    out_ref[...] = lax.rsqrt(1.0 + total)


def _dis_call(counts):
    return pl.pallas_call(
        _dis_body, out_shape=jax.ShapeDtypeStruct((NPAD,), jnp.float32))(counts)


def _hs1_body(x_ref, w_ref, dis_ref, out_ref):
    out_ref[...] = _mm(x_ref[...], w_ref[...]) * dis_ref[...]


def _hs1_call(x, W1, dis_col):
    return pl.pallas_call(
        _hs1_body,
        grid=(NG,),
        in_specs=[_row_spec(), _full_spec((D, D)), _dis_spec()],
        out_specs=_row_spec(),
        out_shape=jax.ShapeDtypeStruct((N_NODES, D), jnp.float32),
    )(x, W1, dis_col)


def _layer2_body(p_ref, hs_ref, dis_ref, b_ref, wa_ref, wb_ref, oa_ref, ob_ref):
    dis = dis_ref[...]
    agg = p_ref[0] + p_ref[1] + hs_ref[...]
    h = jnp.maximum(agg * dis + b_ref[...], 0.0)
    oa_ref[...] = _mm(h, wa_ref[...]) * dis
    ob_ref[...] = _mm(h, wb_ref[...]) * dis


def _layer2_call(p, hs1, dis_col, b1, W2a, W2b):
    return pl.pallas_call(
        _layer2_body,
        grid=(NG,),
        in_specs=[_p_spec(), _row_spec(), _dis_spec(), _full_spec((1, D)),
                  _full_spec((D, D)), _full_spec((D, D))],
        out_specs=[_row_spec(), _row_spec()],
        out_shape=[jax.ShapeDtypeStruct((N_NODES, D), jnp.float32)] * 2,
    )(p, hs1, dis_col, b1, W2a, W2b)


def _layer3_body(pa_ref, pb_ref, hsa_ref, hsb_ref, dis_ref, ba_ref, bb_ref,
                 wa_ref, wb_ref, out_ref):
    dis = dis_ref[...]
    ha = jnp.maximum((pa_ref[0] + pa_ref[1] + hsa_ref[...]) * dis
                     + ba_ref[...], 0.0)
    hb = jnp.maximum((pb_ref[0] + pb_ref[1] + hsb_ref[...]) * dis
                     + bb_ref[...], 0.0)
    out_ref[...] = (_mm(ha, wa_ref[...]) + _mm(hb, wb_ref[...])) * dis


def _layer3_call(pa, pb, hs2a, hs2b, dis_col, b2a, b2b, W3a, W3b):
    return pl.pallas_call(
        _layer3_body,
        grid=(NG,),
        in_specs=[_p_spec(), _p_spec(), _row_spec(), _row_spec(), _dis_spec(),
                  _full_spec((1, D)), _full_spec((1, D)),
                  _full_spec((D, D)), _full_spec((D, D))],
        out_specs=_row_spec(),
        out_shape=jax.ShapeDtypeStruct((N_NODES, D), jnp.float32),
    )(pa, pb, hs2a, hs2b, dis_col, b2a, b2b, W3a, W3b)


def _head_body(p_ref, hs_ref, dis_ref, b_ref, batch_ref, wf1_ref, bf1_ref,
               wf2_ref, bf2_ref, out_ref, seg_acc, cnt_acc):
    i = pl.program_id(0)

    @pl.when(i == 0)
    def _():
        seg_acc[...] = jnp.zeros_like(seg_acc)
        cnt_acc[...] = jnp.zeros_like(cnt_acc)

    dis = dis_ref[...]
    h3 = jnp.maximum((p_ref[0] + p_ref[1] + hs_ref[...]) * dis
                     + b_ref[...], 0.0)
    gids = lax.broadcasted_iota(jnp.int32, (RB, N_GRAPHS), 1)
    onehot_t = (gids == batch_ref[...]).astype(jnp.float32)
    seg_acc[...] += lax.dot_general(
        onehot_t, h3, (((0,), (0,)), ((), ())),
        precision=lax.Precision.HIGHEST,
        preferred_element_type=jnp.float32)
    cnt_acc[...] += jnp.sum(onehot_t, axis=0)[:, None]

    @pl.when(i == NG - 1)
    def _():
        pooled = seg_acc[...] / jnp.maximum(cnt_acc[...], 1.0)
        o1 = jnp.maximum(_mm(pooled, wf1_ref[...]) + bf1_ref[...], 0.0)
        out_ref[...] = _mm(o1, wf2_ref[...]) + bf2_ref[...]


def _head_call(p3, hs3, dis_col, b3, batch2d, Wf1, bf1, Wf2, bf2):
    return pl.pallas_call(
        _head_body,
        grid=(NG,),
        in_specs=[_p_spec(), _row_spec(), _dis_spec(), _full_spec((1, D)),
                  pl.BlockSpec((RB, 1), lambda i: (i, 0)),
                  _full_spec((D, 64)), _full_spec((1, 64)),
                  _full_spec((64, 1)), _full_spec((1, 1))],
        out_specs=pl.BlockSpec((N_GRAPHS, 1), lambda i: (0, 0)),
        out_shape=jax.ShapeDtypeStruct((N_GRAPHS, 1), jnp.float32),
        scratch_shapes=[pltpu.VMEM((N_GRAPHS, D), jnp.float32),
                        pltpu.VMEM((N_GRAPHS, 1), jnp.float32)],
    )(p3, hs3, dis_col, b3, batch2d, Wf1, bf1, Wf2, bf2)


def kernel(x, edge_index, batch, W1, b1, W2, b2, W3, b3, Wf1, bf1, Wf2, bf2):
    f32 = jnp.float32
    src = edge_index[0].astype(jnp.int32)
    dst = edge_index[1].astype(jnp.int32)
    pad = EPAD - N_EDGES
    srcp = jnp.concatenate([src, jnp.zeros((pad,), jnp.int32)])
    pad_dst = N_NODES + jnp.arange(pad, dtype=jnp.int32) % (NPAD - N_NODES)
    dstp = jnp.concatenate([dst, pad_dst])
    src3 = srcp.reshape(NW, NIT, EB)
    dst3 = dstp.reshape(NW, NIT, EB)
    dst2 = dstp.reshape(NW, NIT * EB)

    counts = _sc_counts(dst2)
    dis_flat = _dis_call(counts)
    dis_col = dis_flat.reshape(NPAD, 1)[:N_NODES]

    hs1 = _hs1_call(x, W1, dis_col)
    p1 = _sc_agg(src3, dst3, hs1)
    hs2a, hs2b = _layer2_call(p1, hs1, dis_col, b1.reshape(1, D),
                              W2[:, :D], W2[:, D:])

    p2a = _sc_agg(src3, dst3, hs2a)
    p2b = _sc_agg(src3, dst3, hs2b)
    hs3 = _layer3_call(p2a, p2b, hs2a, hs2b, dis_col,
                       b2[:D].reshape(1, D), b2[D:].reshape(1, D),
                       W3[:D, :], W3[D:, :])

    p3 = _sc_agg(src3, dst3, hs3)
    out = _head_call(p3, hs3, dis_col, b3.reshape(1, D),
                     batch.astype(jnp.int32).reshape(N_NODES, 1),
                     Wf1, bf1.reshape(1, 64), Wf2, bf2.reshape(1, 1))
    return out


# speedup vs baseline: 1.1418x; 1.0340x over previous
import functools

import jax
import jax.numpy as jnp
from jax import lax
from jax.experimental import pallas as pl
from jax.experimental.pallas import tpu as pltpu
from jax.experimental.pallas import tpu_sc as plsc

N_NODES = 10000
N_EDGES = 320000
N_GRAPHS = 64
D = 128

NC = 2
NS = 16
NW = NC * NS
EB = 128
NIT = 80
EPAD = NW * NIT * EB
NPAD = 10240
RPT = NPAD // NS

_mesh = plsc.VectorSubcoreMesh(core_axis_name="c", subcore_axis_name="s")
_sc_params = pltpu.CompilerParams(needs_layout_passes=False)


@functools.partial(
    pl.kernel,
    out_type=jax.ShapeDtypeStruct((NW, NPAD), jnp.float32),
    mesh=_mesh,
    scratch_types=[
        pltpu.VMEM((NIT * EB,), jnp.int32),
        pltpu.VMEM((NPAD,), jnp.float32),
    ],
    compiler_params=_sc_params,
)
def _sc_counts(dst_hbm, out_hbm, dst_v, cnt_v):
    c = lax.axis_index("c")
    s = lax.axis_index("s")
    wid = s * NC + c
    pltpu.sync_copy(dst_hbm.at[wid], dst_v)

    def zero(i, carry):
        cnt_v[pl.ds(i * 16, 16)] = jnp.zeros((16,), jnp.float32)
        return carry

    lax.fori_loop(0, NPAD // 16, zero, 0)

    ones = jnp.ones((16,), jnp.float32)

    def body(k, carry):
        idx = dst_v[pl.ds(k * 16, 16)]
        plsc.addupdate_scatter(cnt_v, [idx], ones)
        return carry

    lax.fori_loop(0, (NIT * EB) // 16, body, 0)
    pltpu.sync_copy(cnt_v, out_hbm.at[wid])


NBUF = 2
WIN = 16
NWIN = NIT // WIN


@functools.partial(
    pl.kernel,
    out_type=jax.ShapeDtypeStruct((NC, NPAD, D), jnp.float32),
    mesh=_mesh,
    scratch_types=[
        pltpu.VMEM((2, WIN, EB), jnp.int32),
        pltpu.VMEM((2, WIN, EB), jnp.int32),
        pltpu.VMEM((EB, D), jnp.float32),
        pltpu.VMEM((EB, D), jnp.float32),
        pltpu.SemaphoreType.DMA,
        pltpu.SemaphoreType.DMA,
        pltpu.SemaphoreType.DMA,
        pltpu.SemaphoreType.DMA,
        pltpu.VMEM_SHARED((NPAD, D), jnp.float32),
    ],
    compiler_params=_sc_params,
)
def _sc_agg(src_hbm, dst_hbm, table_hbm, out_hbm, src_w, dst_w,
            buf0, buf1, sem0, sem1, isem_s, isem_d, acc):
    bufs = (buf0, buf1)
    sems = (sem0, sem1)
    c = lax.axis_index("c")
    s = lax.axis_index("s")
    wid = s * NC + c

    pltpu.sync_copy(src_hbm.at[wid, pl.ds(0, WIN)], src_w.at[0])
    pltpu.sync_copy(dst_hbm.at[wid, pl.ds(0, WIN)], dst_w.at[0])

    def zero(i, carry):
        for cc in range(D // 16):
            buf0[i, pl.ds(cc * 16, 16)] = jnp.zeros((16,), jnp.float32)
        return carry

    lax.fori_loop(0, EB, zero, 0)
    for t in range(RPT // EB):
        pltpu.sync_copy(buf0, acc.at[pl.ds(s * RPT + t * EB, EB)])
    plsc.subcore_barrier()

    for w in range(NWIN):
        p = w % 2
        if w + 1 < NWIN:
            np_ = (w + 1) % 2
            pltpu.async_copy(src_hbm.at[wid, pl.ds((w + 1) * WIN, WIN)],
                             src_w.at[np_], isem_s)
            pltpu.async_copy(dst_hbm.at[wid, pl.ds((w + 1) * WIN, WIN)],
                             dst_w.at[np_], isem_d)
        for b in range(NBUF):
            pltpu.async_copy(table_hbm.at[src_w.at[p, b]], bufs[b], sems[b])

        def body(g, carry, p=p):
            for b in range(NBUF):
                j = g * NBUF + b
                pltpu.make_async_copy(table_hbm.at[src_w.at[p, j]],
                                      bufs[b], sems[b]).wait()
                pltpu.sync_copy(bufs[b], acc.at[dst_w.at[p, j]], add=True)
                nj = j + NBUF

                @pl.when(nj < WIN)
                def _():
                    pltpu.async_copy(table_hbm.at[src_w.at[p, nj]],
                                     bufs[b], sems[b])
            return carry

        lax.fori_loop(0, WIN // NBUF, body, 0)
        if w + 1 < NWIN:
            np_ = (w + 1) % 2
            pltpu.make_async_copy(src_hbm.at[wid, pl.ds((w + 1) * WIN, WIN)],
                                  src_w.at[np_], isem_s).wait()
            pltpu.make_async_copy(dst_hbm.at[wid, pl.ds((w + 1) * WIN, WIN)],
                                  dst_w.at[np_], isem_d).wait()

    plsc.subcore_barrier()
    pltpu.sync_copy(acc.at[pl.ds(s * RPT, RPT)], out_hbm.at[c, pl.ds(s * RPT, RPT)])


RB = 1000
NG = N_NODES // RB


def _mm(a, b):
    return jnp.dot(a, b, preferred_element_type=jnp.float32)


def _row_spec():
    return pl.BlockSpec((RB, D), lambda i: (i, 0))


def _p_spec():
    return pl.BlockSpec((NC, RB, D), lambda i: (0, i, 0))


def _dis_spec():
    return pl.BlockSpec((RB, 1), lambda i: (i, 0))


def _full_spec(shape):
    nd = len(shape)
    return pl.BlockSpec(shape, lambda i: (0,) * nd)


def _dis_body(cnt_ref, out_ref):
    total = jnp.sum(cnt_ref[...], axis=0)
    out_ref[...] = lax.rsqrt(1.0 + total)


def _dis_call(counts):
    return pl.pallas_call(
        _dis_body, out_shape=jax.ShapeDtypeStruct((NPAD,), jnp.float32))(counts)


def _hs1_body(x_ref, w_ref, dis_ref, out_ref):
    out_ref[...] = _mm(x_ref[...], w_ref[...]) * dis_ref[...]


def _hs1_call(x, W1, dis_col):
    return pl.pallas_call(
        _hs1_body,
        grid=(NG,),
        in_specs=[_row_spec(), _full_spec((D, D)), _dis_spec()],
        out_specs=_row_spec(),
        out_shape=jax.ShapeDtypeStruct((N_NODES, D), jnp.float32),
    )(x, W1, dis_col)


def _layer2_body(p_ref, hs_ref, dis_ref, b_ref, wa_ref, wb_ref, oa_ref, ob_ref):
    dis = dis_ref[...]
    agg = p_ref[0] + p_ref[1] + hs_ref[...]
    h = jnp.maximum(agg * dis + b_ref[...], 0.0)
    oa_ref[...] = _mm(h, wa_ref[...]) * dis
    ob_ref[...] = _mm(h, wb_ref[...]) * dis


def _layer2_call(p, hs1, dis_col, b1, W2a, W2b):
    return pl.pallas_call(
        _layer2_body,
        grid=(NG,),
        in_specs=[_p_spec(), _row_spec(), _dis_spec(), _full_spec((1, D)),
                  _full_spec((D, D)), _full_spec((D, D))],
        out_specs=[_row_spec(), _row_spec()],
        out_shape=[jax.ShapeDtypeStruct((N_NODES, D), jnp.float32)] * 2,
    )(p, hs1, dis_col, b1, W2a, W2b)


def _layer3_body(pa_ref, pb_ref, hsa_ref, hsb_ref, dis_ref, ba_ref, bb_ref,
                 wa_ref, wb_ref, out_ref):
    dis = dis_ref[...]
    ha = jnp.maximum((pa_ref[0] + pa_ref[1] + hsa_ref[...]) * dis
                     + ba_ref[...], 0.0)
    hb = jnp.maximum((pb_ref[0] + pb_ref[1] + hsb_ref[...]) * dis
                     + bb_ref[...], 0.0)
    out_ref[...] = (_mm(ha, wa_ref[...]) + _mm(hb, wb_ref[...])) * dis


def _layer3_call(pa, pb, hs2a, hs2b, dis_col, b2a, b2b, W3a, W3b):
    return pl.pallas_call(
        _layer3_body,
        grid=(NG,),
        in_specs=[_p_spec(), _p_spec(), _row_spec(), _row_spec(), _dis_spec(),
                  _full_spec((1, D)), _full_spec((1, D)),
                  _full_spec((D, D)), _full_spec((D, D))],
        out_specs=_row_spec(),
        out_shape=jax.ShapeDtypeStruct((N_NODES, D), jnp.float32),
    )(pa, pb, hs2a, hs2b, dis_col, b2a, b2b, W3a, W3b)


def _head_body(p_ref, hs_ref, dis_ref, b_ref, batch_ref, wf1_ref, bf1_ref,
               wf2_ref, bf2_ref, out_ref, seg_acc, cnt_acc):
    i = pl.program_id(0)

    @pl.when(i == 0)
    def _():
        seg_acc[...] = jnp.zeros_like(seg_acc)
        cnt_acc[...] = jnp.zeros_like(cnt_acc)

    dis = dis_ref[...]
    h3 = jnp.maximum((p_ref[0] + p_ref[1] + hs_ref[...]) * dis
                     + b_ref[...], 0.0)
    gids = lax.broadcasted_iota(jnp.int32, (RB, N_GRAPHS), 1)
    onehot_t = (gids == batch_ref[...]).astype(jnp.float32)
    seg_acc[...] += lax.dot_general(
        onehot_t, h3, (((0,), (0,)), ((), ())),
        precision=lax.Precision.HIGHEST,
        preferred_element_type=jnp.float32)
    cnt_acc[...] += jnp.sum(onehot_t, axis=0)[:, None]

    @pl.when(i == NG - 1)
    def _():
        pooled = seg_acc[...] / jnp.maximum(cnt_acc[...], 1.0)
        o1 = jnp.maximum(_mm(pooled, wf1_ref[...]) + bf1_ref[...], 0.0)
        out_ref[...] = _mm(o1, wf2_ref[...]) + bf2_ref[...]


def _head_call(p3, hs3, dis_col, b3, batch2d, Wf1, bf1, Wf2, bf2):
    return pl.pallas_call(
        _head_body,
        grid=(NG,),
        in_specs=[_p_spec(), _row_spec(), _dis_spec(), _full_spec((1, D)),
                  pl.BlockSpec((RB, 1), lambda i: (i, 0)),
                  _full_spec((D, 64)), _full_spec((1, 64)),
                  _full_spec((64, 1)), _full_spec((1, 1))],
        out_specs=pl.BlockSpec((N_GRAPHS, 1), lambda i: (0, 0)),
        out_shape=jax.ShapeDtypeStruct((N_GRAPHS, 1), jnp.float32),
        scratch_shapes=[pltpu.VMEM((N_GRAPHS, D), jnp.float32),
                        pltpu.VMEM((N_GRAPHS, 1), jnp.float32)],
    )(p3, hs3, dis_col, b3, batch2d, Wf1, bf1, Wf2, bf2)


def kernel(x, edge_index, batch, W1, b1, W2, b2, W3, b3, Wf1, bf1, Wf2, bf2):
    f32 = jnp.float32
    src = edge_index[0].astype(jnp.int32)
    dst = edge_index[1].astype(jnp.int32)
    pad = EPAD - N_EDGES
    srcp = jnp.concatenate([src, jnp.zeros((pad,), jnp.int32)])
    pad_dst = N_NODES + jnp.arange(pad, dtype=jnp.int32) % (NPAD - N_NODES)
    dstp = jnp.concatenate([dst, pad_dst])
    src3 = srcp.reshape(NW, NIT, EB)
    dst3 = dstp.reshape(NW, NIT, EB)
    dst2 = dstp.reshape(NW, NIT * EB)

    counts = _sc_counts(dst2)
    dis_flat = _dis_call(counts)
    dis_col = dis_flat.reshape(NPAD, 1)[:N_NODES]

    hs1 = _hs1_call(x, W1, dis_col)
    p1 = _sc_agg(src3, dst3, hs1)
    hs2a, hs2b = _layer2_call(p1, hs1, dis_col, b1.reshape(1, D),
                              W2[:, :D], W2[:, D:])

    p2a = _sc_agg(src3, dst3, hs2a)
    p2b = _sc_agg(src3, dst3, hs2b)
    hs3 = _layer3_call(p2a, p2b, hs2a, hs2b, dis_col,
                       b2[:D].reshape(1, D), b2[D:].reshape(1, D),
                       W3[:D, :], W3[D:, :])

    p3 = _sc_agg(src3, dst3, hs3)
    out = _head_call(p3, hs3, dis_col, b3.reshape(1, D),
                     batch.astype(jnp.int32).reshape(N_NODES, 1),
                     Wf1, bf1.reshape(1, 64), Wf2, bf2.reshape(1, 1))
    return out
